# baseline (speedup 1.0000x reference)
import functools

import jax
import jax.numpy as jnp
from jax import lax
from jax.experimental import pallas as pl
from jax.experimental.pallas import tpu as pltpu
from jax.experimental.pallas import tpu_sc as plsc

NU_ = 50000
NI_ = 50000
N_ = NU_ + NI_
NP2 = 100096
E_ = 1600000
NC, NS = 2, 16
NW = NC * NS
W_ = 1024
KROWS = W_ // 128
WIN = 50
EPAD = NW * WIN * W_
TPT = NP2 // NS
ZCH = 16
ZROWS = TPT // ZCH

_mesh = plsc.VectorSubcoreMesh(
    core_axis_name="c", subcore_axis_name="s", num_cores=NC, num_subcores=NS)


@functools.partial(
    pl.kernel,
    out_type=jax.ShapeDtypeStruct((2 * NP2, 16), jnp.float32),
    mesh=_mesh,
    scratch_types=[
        pltpu.VMEM((W_,), jnp.int32),
        pltpu.VMEM((W_,), jnp.int32),
        pltpu.VMEM((W_, 16), jnp.float32),
        pltpu.VMEM((ZROWS, 16), jnp.float32),
        pltpu.VMEM_SHARED((NP2, 16), jnp.float32),
        pltpu.SemaphoreType.DMA,
        pltpu.SemaphoreType.DMA,
    ],
    compiler_params=pltpu.CompilerParams(use_tc_tiling_on_sc=False),
)
def _prop(zj, src2, dst2, oj,
          idxs_v, idxd_v, msg_v, zbuf_v, acc_sh, gsem, ssem):
    c = lax.axis_index("c")
    s = lax.axis_index("s")
    wid = s * NC + c

    def _zb(i, carry):
        zbuf_v[i, :] = jnp.zeros((16,), jnp.float32)
        return carry

    lax.fori_loop(0, ZROWS, _zb, 0)
    for q in range(ZCH):
        pltpu.sync_copy(zbuf_v, acc_sh.at[pl.ds(s * TPT + q * ZROWS, ZROWS)])
    plsc.subcore_barrier()

    def _win(i, carry):
        ebase = wid * (WIN * W_) + i * W_
        pltpu.sync_copy(src2.at[pl.ds(ebase, W_)], idxs_v)
        pltpu.sync_copy(dst2.at[pl.ds(ebase, W_)], idxd_v)
        pltpu.async_copy(zj.at[idxs_v], msg_v, gsem).wait()
        pltpu.async_copy(msg_v, acc_sh.at[idxd_v], ssem, add=True).wait()
        return carry

    lax.fori_loop(0, WIN, _win, 0)
    plsc.subcore_barrier()
    pltpu.sync_copy(acc_sh.at[pl.ds(s * TPT, TPT)],
                    oj.at[pl.ds(c * NP2 + s * TPT, TPT)])


def _fuse_body(u0, u1, u2, i0, i1, i2, ct, w1, b1, w2, b2, ug, uf_o, itf_o):
    h = jnp.maximum(ct[...] @ w1[...] + b1[...], 0.0)
    g = jax.nn.softmax(h @ w2[...] + b2[...], axis=-1)
    itf_o[...] = (g[:, 0:1] * i0[...] + g[:, 1:2] * i1[...]
                  + g[:, 2:3] * i2[...])
    uf_o[...] = (ug[0, 0] * u0[...] + ug[0, 1] * u1[...]
                 + ug[0, 2] * u2[...])


_BR = 400


def _fuse(u0, u1, u2, i0, i1, i2, ct, w1, b1, w2, b2, ug):
    row = pl.BlockSpec((_BR, 32), lambda b: (b, 0))
    full = lambda shp: pl.BlockSpec(shp, lambda b: (0, 0))
    return pl.pallas_call(
        _fuse_body,
        grid=(NI_ // _BR,),
        in_specs=[row, row, row, row, row, row,
                  pl.BlockSpec((_BR, 128), lambda b: (b, 0)),
                  full((128, 64)), full((1, 64)), full((64, 3)),
                  full((1, 3)), full((1, 3))],
        out_specs=[row, row],
        out_shape=[jax.ShapeDtypeStruct((NU_, 32), jnp.float32),
                   jax.ShapeDtypeStruct((NI_, 32), jnp.float32)],
    )(u0, u1, u2, i0, i1, i2, ct, w1, b1, w2, b2, ug)


_BB = 4096 // NW


@functools.partial(
    pl.kernel,
    out_type=[jax.ShapeDtypeStruct((4096, 32), jnp.float32)] * 3,
    mesh=_mesh,
    scratch_types=[
        pltpu.VMEM((_BB,), jnp.int32),
        pltpu.VMEM((_BB, 32), jnp.float32),
        pltpu.SemaphoreType.DMA,
    ],
    compiler_params=pltpu.CompilerParams(use_tc_tiling_on_sc=False),
)
def _bpr(uf, itf, us, ps, ns, oue, ope, one, idx_v, row_v, sem):
    c = lax.axis_index("c")
    s = lax.axis_index("s")
    wid = s * NC + c
    base = wid * _BB
    for tab, iv, ov in ((uf, us, oue), (itf, ps, ope), (itf, ns, one)):
        pltpu.sync_copy(iv.at[pl.ds(base, _BB)], idx_v)
        pltpu.async_copy(tab.at[idx_v], row_v, sem).wait()
        pltpu.sync_copy(row_v, ov.at[pl.ds(base, _BB)])


def kernel(u_id, i_id, u_tx, i_tx, u_im, i_im, W1, b1, W2, b2,
           user_modal_logits, item_text_feats, item_image_feats,
           edge_w, edge_src, edge_dst, users, pos_items, neg_items):
    deg = jnp.zeros((N_,), jnp.float32).at[edge_src].add(1.0).at[edge_dst].add(1.0)
    sca = lax.rsqrt(jnp.maximum(deg, 1.0))
    sp = jnp.pad(sca, (0, NP2 - N_))[:, None]

    a0 = jnp.concatenate([
        jnp.concatenate([u_id, i_id], 0),
        jnp.concatenate([u_tx, i_tx], 0),
        jnp.concatenate([u_im, i_im], 0)], 1)
    a0 = jnp.pad(a0, ((0, NP2 - N_), (0, 0)))
    a0c = [a0[:, 16 * j:16 * j + 16] for j in range(6)]

    padi = jnp.full((EPAD - E_,), N_, jnp.int32)
    src2 = jnp.concatenate([edge_src, padi])
    dst2 = jnp.concatenate([edge_dst, padi])

    z = [sp * x for x in a0c]
    outs1 = [_prop(zj, src2, dst2) for zj in z]
    a1c = [sp * (o[:NP2] + o[NP2:]) for o in outs1]
    z1 = [sp * x for x in a1c]
    outs2 = [_prop(zj, src2, dst2) for zj in z1]
    a2c = [sp * (o[:NP2] + o[NP2:]) for o in outs2]

    mc = [(x0 + x1 + x2) * (1.0 / 3.0) for x0, x1, x2 in zip(a0c, a1c, a2c)]
    u_m = [jnp.concatenate([mc[2 * m][:NU_], mc[2 * m + 1][:NU_]], 1)
           for m in range(3)]
    i_m = [jnp.concatenate([mc[2 * m][NU_:N_], mc[2 * m + 1][NU_:N_]], 1)
           for m in range(3)]

    content = jnp.concatenate([item_text_feats, item_image_feats], 1)
    ug = jax.nn.softmax(user_modal_logits)[None, :]
    uf, itf = _fuse(u_m[0], u_m[1], u_m[2], i_m[0], i_m[1], i_m[2], content,
                    W1, b1[None, :], W2, b2[None, :], ug)

    u_e, pos_e, neg_e = _bpr(uf, itf, users, pos_items, neg_items)
    return (u_e, pos_e, neg_e)


# speedup vs baseline: 1.1030x; 1.1030x over previous
import functools

import jax
import jax.numpy as jnp
from jax import lax
from jax.experimental import pallas as pl
from jax.experimental.pallas import tpu as pltpu
from jax.experimental.pallas import tpu_sc as plsc

NU_ = 50000
NI_ = 50000
N_ = NU_ + NI_
NP2 = 100096
E_ = 1600000
NC, NS = 2, 16
NW = NC * NS
W_ = 800
WIN = 64
EPAD = NW * WIN * W_
TPT = NP2 // NS

_mesh = plsc.VectorSubcoreMesh(
    core_axis_name="c", subcore_axis_name="s", num_cores=NC, num_subcores=NS)


@functools.partial(
    pl.kernel,
    out_type=jax.ShapeDtypeStruct((2 * NP2, 16), jnp.float32),
    mesh=_mesh,
    scratch_types=[
        pltpu.VMEM((2, W_), jnp.int32),
        pltpu.VMEM((2, W_), jnp.int32),
        pltpu.VMEM((2, W_, 16), jnp.float32),
        pltpu.VMEM_SHARED((NP2, 16), jnp.float32),
        pltpu.SemaphoreType.DMA,
        pltpu.SemaphoreType.DMA,
        pltpu.SemaphoreType.DMA,
        pltpu.SemaphoreType.DMA,
    ],
    compiler_params=pltpu.CompilerParams(use_tc_tiling_on_sc=False),
)
def _prop(zj, src2, dst2, oj,
          idxs_v, idxd_v, msg_v, acc_sh, gsem0, gsem1, ssem0, ssem1):
    c = lax.axis_index("c")
    s = lax.axis_index("s")
    wid = s * NC + c
    gsems = (gsem0, gsem1)
    ssems = (ssem0, ssem1)

    def _zb(i, carry):
        msg_v[0, i, :] = jnp.zeros((16,), jnp.float32)
        return carry

    lax.fori_loop(0, W_, _zb, 0)
    for q in range(7):
        pltpu.sync_copy(msg_v.at[0],
                        acc_sh.at[pl.ds(s * TPT + q * W_, W_)])
    pltpu.sync_copy(msg_v.at[0].at[pl.ds(0, TPT - 7 * W_)],
                    acc_sh.at[pl.ds(s * TPT + 7 * W_, TPT - 7 * W_)])
    plsc.subcore_barrier()

    def _pair(ii, carry):
        for b in range(2):
            i = ii * 2 + b
            nb = 1 - b

            @pl.when(i >= 2)
            def _():
                pltpu.make_async_copy(msg_v.at[b], acc_sh.at[idxd_v.at[b]],
                                      ssems[b]).wait()

            ebase = wid * (WIN * W_) + i * W_
            pltpu.sync_copy(src2.at[pl.ds(ebase, W_)], idxs_v.at[b])
            pltpu.sync_copy(dst2.at[pl.ds(ebase, W_)], idxd_v.at[b])
            pltpu.async_copy(zj.at[idxs_v.at[b]], msg_v.at[b], gsems[b])

            @pl.when(i >= 1)
            def _():
                pltpu.make_async_copy(zj.at[idxs_v.at[nb]], msg_v.at[nb],
                                      gsems[nb]).wait()
                pltpu.async_copy(msg_v.at[nb], acc_sh.at[idxd_v.at[nb]],
                                 ssems[nb], add=True)
        return carry

    lax.fori_loop(0, WIN // 2, _pair, 0)
    pltpu.make_async_copy(zj.at[idxs_v.at[1]], msg_v.at[1], gsems[1]).wait()
    pltpu.async_copy(msg_v.at[1], acc_sh.at[idxd_v.at[1]], ssems[1], add=True)
    pltpu.make_async_copy(msg_v.at[0], acc_sh.at[idxd_v.at[0]], ssems[0]).wait()
    pltpu.make_async_copy(msg_v.at[1], acc_sh.at[idxd_v.at[1]], ssems[1]).wait()
    plsc.subcore_barrier()
    pltpu.sync_copy(acc_sh.at[pl.ds(s * TPT, TPT)],
                    oj.at[pl.ds(c * NP2 + s * TPT, TPT)])


def _fuse_body(u0, u1, u2, i0, i1, i2, ct, w1, b1, w2, b2, ug, uf_o, itf_o):
    h = jnp.maximum(ct[...] @ w1[...] + b1[...], 0.0)
    g = jax.nn.softmax(h @ w2[...] + b2[...], axis=-1)
    itf_o[...] = (g[:, 0:1] * i0[...] + g[:, 1:2] * i1[...]
                  + g[:, 2:3] * i2[...])
    uf_o[...] = (ug[0, 0] * u0[...] + ug[0, 1] * u1[...]
                 + ug[0, 2] * u2[...])


_BR = 400


def _fuse(u0, u1, u2, i0, i1, i2, ct, w1, b1, w2, b2, ug):
    row = pl.BlockSpec((_BR, 32), lambda b: (b, 0))
    full = lambda shp: pl.BlockSpec(shp, lambda b: (0, 0))
    return pl.pallas_call(
        _fuse_body,
        grid=(NI_ // _BR,),
        in_specs=[row, row, row, row, row, row,
                  pl.BlockSpec((_BR, 128), lambda b: (b, 0)),
                  full((128, 64)), full((1, 64)), full((64, 3)),
                  full((1, 3)), full((1, 3))],
        out_specs=[row, row],
        out_shape=[jax.ShapeDtypeStruct((NU_, 32), jnp.float32),
                   jax.ShapeDtypeStruct((NI_, 32), jnp.float32)],
    )(u0, u1, u2, i0, i1, i2, ct, w1, b1, w2, b2, ug)


_BB = 4096 // NW


@functools.partial(
    pl.kernel,
    out_type=[jax.ShapeDtypeStruct((4096, 32), jnp.float32)] * 3,
    mesh=_mesh,
    scratch_types=[
        pltpu.VMEM((_BB,), jnp.int32),
        pltpu.VMEM((_BB, 32), jnp.float32),
        pltpu.SemaphoreType.DMA,
    ],
    compiler_params=pltpu.CompilerParams(use_tc_tiling_on_sc=False),
)
def _bpr(uf, itf, us, ps, ns, oue, ope, one, idx_v, row_v, sem):
    c = lax.axis_index("c")
    s = lax.axis_index("s")
    wid = s * NC + c
    base = wid * _BB
    for tab, iv, ov in ((uf, us, oue), (itf, ps, ope), (itf, ns, one)):
        pltpu.sync_copy(iv.at[pl.ds(base, _BB)], idx_v)
        pltpu.async_copy(tab.at[idx_v], row_v, sem).wait()
        pltpu.sync_copy(row_v, ov.at[pl.ds(base, _BB)])


def kernel(u_id, i_id, u_tx, i_tx, u_im, i_im, W1, b1, W2, b2,
           user_modal_logits, item_text_feats, item_image_feats,
           edge_w, edge_src, edge_dst, users, pos_items, neg_items):
    deg = jnp.zeros((N_,), jnp.float32).at[edge_src].add(1.0).at[edge_dst].add(1.0)
    sca = lax.rsqrt(jnp.maximum(deg, 1.0))
    sp = jnp.pad(sca, (0, NP2 - N_))[:, None]

    a0 = jnp.concatenate([
        jnp.concatenate([u_id, i_id], 0),
        jnp.concatenate([u_tx, i_tx], 0),
        jnp.concatenate([u_im, i_im], 0)], 1)
    a0 = jnp.pad(a0, ((0, NP2 - N_), (0, 0)))
    a0c = [a0[:, 16 * j:16 * j + 16] for j in range(6)]

    padi = jnp.full((EPAD - E_,), N_, jnp.int32)
    src2 = jnp.concatenate([edge_src, padi])
    dst2 = jnp.concatenate([edge_dst, padi])

    z = [sp * x for x in a0c]
    outs1 = [_prop(zj, src2, dst2) for zj in z]
    a1c = [sp * (o[:NP2] + o[NP2:]) for o in outs1]
    z1 = [sp * x for x in a1c]
    outs2 = [_prop(zj, src2, dst2) for zj in z1]
    a2c = [sp * (o[:NP2] + o[NP2:]) for o in outs2]

    mc = [(x0 + x1 + x2) * (1.0 / 3.0) for x0, x1, x2 in zip(a0c, a1c, a2c)]
    u_m = [jnp.concatenate([mc[2 * m][:NU_], mc[2 * m + 1][:NU_]], 1)
           for m in range(3)]
    i_m = [jnp.concatenate([mc[2 * m][NU_:N_], mc[2 * m + 1][NU_:N_]], 1)
           for m in range(3)]

    content = jnp.concatenate([item_text_feats, item_image_feats], 1)
    ug = jax.nn.softmax(user_modal_logits)[None, :]
    uf, itf = _fuse(u_m[0], u_m[1], u_m[2], i_m[0], i_m[1], i_m[2], content,
                    W1, b1[None, :], W2, b2[None, :], ug)

    u_e, pos_e, neg_e = _bpr(uf, itf, users, pos_items, neg_items)
    return (u_e, pos_e, neg_e)
